# baseline (speedup 1.0000x reference)
import functools

import jax
import jax.numpy as jnp
from jax import lax
from jax.experimental import pallas as pl
from jax.experimental.pallas import tpu as pltpu
from jax.experimental.pallas import tpu_sc as plsc


T_SC = 1024
BLOCK_TC = 1024
NW = 32
TOKW = T_SC // NW
XC = 4


def _tc_gate(x_ref, wt_ref, b_ref, out_ref):
    logits = jnp.dot(x_ref[...], wt_ref[...],
                     preferred_element_type=jnp.float32) + b_ref[...]
    m = jnp.max(logits, axis=-1, keepdims=True)
    e = jnp.exp(logits - m)
    out_ref[...] = e / jnp.sum(e, axis=-1, keepdims=True)


def _sc_gate(t_tc, x_hbm, wt_hbm, b_hbm, out_hbm, wt_v, xbuf, obuf, b_v):
    wid = lax.axis_index("s") * 2 + lax.axis_index("c")
    base = t_tc + wid * TOKW
    pltpu.sync_copy(wt_hbm, wt_v)
    pltpu.sync_copy(b_hbm, b_v)
    bvec = b_v[...]
    zero = jnp.zeros((16,), jnp.float32)

    def chunk_body(ck, carry0):
        pltpu.sync_copy(x_hbm.at[pl.ds(base + ck * XC, XC), :], xbuf)

        def token_body(u, carry):
            def dim_body(i, accs):
                a0, a1, a2, a3 = accs
                d = i * 16
                xv = xbuf[u, pl.ds(d, 16)]
                r = i * 2
                for k in range(16):
                    w = wt_v[r + k // 8, pl.ds((k % 8) * 16, 16)]
                    prod = xv[k] * w
                    if k % 4 == 0:
                        a0 = a0 + prod
                    elif k % 4 == 1:
                        a1 = a1 + prod
                    elif k % 4 == 2:
                        a2 = a2 + prod
                    else:
                        a3 = a3 + prod
                return (a0, a1, a2, a3)

            a0, a1, a2, a3 = lax.fori_loop(0, 128, dim_body,
                                           (zero, zero, zero, zero))
            acc = (a0 + a1) + (a2 + a3) + bvec
            m = jnp.max(acc)
            e = jnp.exp(acc - m)
            obuf[ck * XC + u] = e / jnp.sum(e)
            return carry

        lax.fori_loop(0, XC, token_body, 0)
        return carry0

    lax.fori_loop(0, TOKW // XC, chunk_body, 0)
    pltpu.sync_copy(obuf, out_hbm.at[pl.ds(wid * TOKW, TOKW), :])


def kernel(payload_tensor, W, b):
    B, S, D = payload_tensor.shape
    C = W.shape[0]
    T = B * S
    t_tc = T - T_SC
    x2 = payload_tensor.reshape(T, D)
    wt = W.T
    wt_packed = W.T.reshape(D // 8, 8 * C)
    b2 = b.reshape(1, C)

    out_tc = pl.pallas_call(
        _tc_gate,
        grid=(t_tc // BLOCK_TC,),
        in_specs=[
            pl.BlockSpec((BLOCK_TC, D), lambda i: (i, 0)),
            pl.BlockSpec((D, C), lambda i: (0, 0)),
            pl.BlockSpec((1, C), lambda i: (0, 0)),
        ],
        out_specs=pl.BlockSpec((BLOCK_TC, C), lambda i: (i, 0)),
        out_shape=jax.ShapeDtypeStruct((t_tc, C), jnp.float32),
    )(x2, wt, b2)

    sc_call = functools.partial(
        pl.kernel,
        mesh=plsc.VectorSubcoreMesh(core_axis_name="c", subcore_axis_name="s"),
        out_type=jax.ShapeDtypeStruct((T_SC, C), jnp.float32),
        scratch_types=[
            pltpu.VMEM((D // 8, 8 * C), jnp.float32),
            pltpu.VMEM((XC, D), jnp.float32),
            pltpu.VMEM((TOKW, C), jnp.float32),
            pltpu.VMEM((C,), jnp.float32),
        ],
        compiler_params=pltpu.CompilerParams(needs_layout_passes=False),
    )(functools.partial(_sc_gate, t_tc))
    out_sc = sc_call(x2, wt_packed, b)

    out = jnp.concatenate([out_tc, out_sc], axis=0)
    return out.reshape(B, S, C)


# speedup vs baseline: 1.0053x; 1.0053x over previous
import functools

import jax
import jax.numpy as jnp
from jax import lax
from jax.experimental import pallas as pl
from jax.experimental.pallas import tpu as pltpu
from jax.experimental.pallas import tpu_sc as plsc


T_SC = 1024
BLOCK_TC = 1024
NW = 32
TOKW = T_SC // NW
XC = 4


def _tc_gate(x_ref, wt_ref, b_ref, out_ref):
    logits = jnp.dot(x_ref[...], wt_ref[...],
                     preferred_element_type=jnp.float32) + b_ref[...]
    m = jnp.max(logits, axis=-1, keepdims=True)
    e = jnp.exp(logits - m)
    out_ref[...] = e / jnp.sum(e, axis=-1, keepdims=True)


def _sc_gate(t_tc, x_hbm, wt_hbm, b_hbm, out_hbm, wt_v, xbuf, obuf, b_v):
    wid = lax.axis_index("s") * 2 + lax.axis_index("c")
    base = t_tc + wid * TOKW
    pltpu.sync_copy(wt_hbm, wt_v)
    pltpu.sync_copy(b_hbm, b_v)
    bvec = b_v[...]
    zero = jnp.zeros((16,), jnp.float32)

    def chunk_body(ck, carry0):
        pltpu.sync_copy(x_hbm.at[pl.ds(base + ck * XC, XC), :], xbuf)

        def token_body(u, carry):
            def dim_body(i, accs):
                a0, a1, a2, a3 = accs
                d = i * 16
                xv = xbuf[u, pl.ds(d, 16)]
                r = i * 2
                for k in range(16):
                    w = wt_v[r + k // 8, pl.ds((k % 8) * 16, 16)]
                    prod = xv[k] * w
                    if k % 4 == 0:
                        a0 = a0 + prod
                    elif k % 4 == 1:
                        a1 = a1 + prod
                    elif k % 4 == 2:
                        a2 = a2 + prod
                    else:
                        a3 = a3 + prod
                return (a0, a1, a2, a3)

            a0, a1, a2, a3 = lax.fori_loop(0, 128, dim_body,
                                           (zero, zero, zero, zero))
            acc = (a0 + a1) + (a2 + a3) + bvec
            m = jnp.max(acc)
            e = jnp.exp(acc - m)
            obuf[ck * XC + u] = e / jnp.sum(e)
            return carry

        lax.fori_loop(0, XC, token_body, 0)
        return carry0

    lax.fori_loop(0, TOKW // XC, chunk_body, 0)
    pltpu.sync_copy(obuf, out_hbm.at[pl.ds(wid * TOKW, TOKW), :])


def kernel(payload_tensor, W, b):
    B, S, D = payload_tensor.shape
    C = W.shape[0]
    T = B * S
    t_tc = T - T_SC
    x2 = payload_tensor.reshape(T, D)
    wt = W.T
    wt_packed = W.T.reshape(D // 8, 8 * C)
    b2 = b.reshape(1, C)

    sc_call = functools.partial(
        pl.kernel,
        mesh=plsc.VectorSubcoreMesh(core_axis_name="c", subcore_axis_name="s"),
        out_type=jax.ShapeDtypeStruct((T_SC, C), jnp.float32),
        scratch_types=[
            pltpu.VMEM((D // 8, 8 * C), jnp.float32),
            pltpu.VMEM((XC, D), jnp.float32),
            pltpu.VMEM((TOKW, C), jnp.float32),
            pltpu.VMEM((C,), jnp.float32),
        ],
        compiler_params=pltpu.CompilerParams(needs_layout_passes=False),
    )(functools.partial(_sc_gate, t_tc))
    out_sc = sc_call(x2, wt_packed, b)

    out_tc = pl.pallas_call(
        _tc_gate,
        grid=(t_tc // BLOCK_TC,),
        in_specs=[
            pl.BlockSpec((BLOCK_TC, D), lambda i: (i, 0)),
            pl.BlockSpec((D, C), lambda i: (0, 0)),
            pl.BlockSpec((1, C), lambda i: (0, 0)),
        ],
        out_specs=pl.BlockSpec((BLOCK_TC, C), lambda i: (i, 0)),
        out_shape=jax.ShapeDtypeStruct((t_tc, C), jnp.float32),
    )(x2, wt, b2)

    out = jnp.concatenate([out_tc, out_sc], axis=0)
    return out.reshape(B, S, C)
